# Initial kernel scaffold; baseline (speedup 1.0000x reference)
import jax
import jax.numpy as jnp
from jax.experimental import pallas as pl


def kernel(vid, Wq, bq, Wk, bk, Wv, bv, Wp, bp):
    raise NotImplementedError("write your pallas kernel here")


# speedup vs baseline: 21.0481x; 21.0481x over previous
import functools

import jax
import jax.numpy as jnp
from jax.experimental import pallas as pl

NHEADS = 4
KS = 64
ROW_BLOCK = 1024


def _qkv_kernel(x_ref, wq_ref, bq_ref, wk_ref, bk_ref, wv_ref, bv_ref,
                q_ref, k_ref, v_ref, *, scale):
    x = x_ref[...]
    q_ref[0] = (jnp.dot(x, wq_ref[0], preferred_element_type=jnp.float32)
                + bq_ref[0]) * scale
    k_ref[0] = jnp.dot(x, wk_ref[0],
                       preferred_element_type=jnp.float32) + bk_ref[0]
    v_ref[0] = jnp.dot(x, wv_ref[0],
                       preferred_element_type=jnp.float32) + bv_ref[0]


def _attn_kernel(q_ref, k_ref, v_ref, wp_ref, bp_ref, out_ref):
    h = pl.program_id(1)
    d = jnp.dot(q_ref[0], k_ref[0].T, preferred_element_type=jnp.float32)

    ib = jax.lax.bitcast_convert_type(d, jnp.int32)
    keys = jnp.where(ib >= 0, ib, ib ^ jnp.int32(0x7FFFFFFF))

    lo = jnp.min(keys, axis=1, keepdims=True)
    hi = jnp.max(keys, axis=1, keepdims=True) + 1

    def body(_, carry):
        lo, hi = carry
        mid = (lo & hi) + ((lo ^ hi) >> 1)
        cnt = jnp.sum((keys >= mid).astype(jnp.int32), axis=1, keepdims=True)
        ge = cnt >= KS
        return jnp.where(ge, mid, lo), jnp.where(ge, hi, mid)

    lo, _ = jax.lax.fori_loop(0, 32, body, (lo, hi))

    mask = keys >= lo
    rowmax = jnp.max(d, axis=1, keepdims=True)
    e = jnp.where(mask, jnp.exp(d - rowmax), 0.0)
    w = e / jnp.sum(e, axis=1, keepdims=True)
    head_out = jnp.dot(w, v_ref[0], preferred_element_type=jnp.float32)
    contrib = jnp.dot(head_out, wp_ref[0], preferred_element_type=jnp.float32)

    @pl.when(h == 0)
    def _():
        out_ref[...] = contrib + bp_ref[...]

    @pl.when(h != 0)
    def _():
        out_ref[...] += contrib


def kernel(vid, Wq, bq, Wk, bk, Wv, bv, Wp, bp):
    Bv, Tv, Cv, Hv, Wd = vid.shape
    N = Tv * Hv * Wd
    dh = Cv // NHEADS
    scale = dh ** -0.5
    x = vid.transpose(0, 1, 3, 4, 2).reshape(N, Cv)

    def col_heads(w):
        return w.reshape(Cv, NHEADS, dh).transpose(1, 0, 2)

    def bias_heads(b):
        return b.reshape(NHEADS, 1, dh)

    q, k, v = pl.pallas_call(
        functools.partial(_qkv_kernel, scale=scale),
        grid=(NHEADS,),
        in_specs=[
            pl.BlockSpec((N, Cv), lambda h: (0, 0)),
            pl.BlockSpec((1, Cv, dh), lambda h: (h, 0, 0)),
            pl.BlockSpec((1, 1, dh), lambda h: (h, 0, 0)),
            pl.BlockSpec((1, Cv, dh), lambda h: (h, 0, 0)),
            pl.BlockSpec((1, 1, dh), lambda h: (h, 0, 0)),
            pl.BlockSpec((1, Cv, dh), lambda h: (h, 0, 0)),
            pl.BlockSpec((1, 1, dh), lambda h: (h, 0, 0)),
        ],
        out_specs=[pl.BlockSpec((1, N, dh), lambda h: (h, 0, 0))] * 3,
        out_shape=[jax.ShapeDtypeStruct((NHEADS, N, dh), jnp.float32)] * 3,
    )(x, col_heads(Wq), bias_heads(bq), col_heads(Wk), bias_heads(bk),
      col_heads(Wv), bias_heads(bv))

    nrb = N // ROW_BLOCK
    out = pl.pallas_call(
        _attn_kernel,
        grid=(nrb, NHEADS),
        in_specs=[
            pl.BlockSpec((1, ROW_BLOCK, dh), lambda rb, h: (h, rb, 0)),
            pl.BlockSpec((1, N, dh), lambda rb, h: (h, 0, 0)),
            pl.BlockSpec((1, N, dh), lambda rb, h: (h, 0, 0)),
            pl.BlockSpec((1, dh, Cv), lambda rb, h: (h, 0, 0)),
            pl.BlockSpec((1, Cv), lambda rb, h: (0, 0)),
        ],
        out_specs=pl.BlockSpec((ROW_BLOCK, Cv), lambda rb, h: (rb, 0)),
        out_shape=jax.ShapeDtypeStruct((N, Cv), jnp.float32),
    )(q, k, v, Wp.reshape(NHEADS, dh, Cv), bp.reshape(1, Cv))

    return out.reshape(Bv, Tv, Hv, Wd, Cv).transpose(0, 1, 4, 2, 3)


# speedup vs baseline: 22.4761x; 1.0678x over previous
import functools

import jax
import jax.numpy as jnp
from jax.experimental import pallas as pl

NHEADS = 4
KS = 64
ROW_BLOCK = 1024


def _qkv_kernel(x_ref, wq_ref, bq_ref, wk_ref, bk_ref, wv_ref, bv_ref,
                q_ref, k_ref, v_ref, *, scale):
    x = x_ref[...]
    q_ref[0] = (jnp.dot(x, wq_ref[0], preferred_element_type=jnp.float32)
                + bq_ref[0]) * scale
    k_ref[0] = jnp.dot(x, wk_ref[0],
                       preferred_element_type=jnp.float32) + bk_ref[0]
    v_ref[0] = jnp.dot(x, wv_ref[0],
                       preferred_element_type=jnp.float32) + bv_ref[0]


def _attn_kernel(q_ref, k_ref, v_ref, wp_ref, bp_ref, out_ref):
    h = pl.program_id(1)
    d = jnp.dot(q_ref[0], k_ref[0].T, preferred_element_type=jnp.float32)

    ib = jax.lax.bitcast_convert_type(d, jnp.int32)
    keys = jnp.where(ib >= 0, ib, ib ^ jnp.int32(0x7FFFFFFF))

    lo = jnp.min(keys, axis=1, keepdims=True)
    hi = jnp.max(keys, axis=1, keepdims=True) + 1

    def cond(carry):
        lo, hi = carry
        return jnp.any(hi > lo + 1)

    def body(carry):
        lo, hi = carry
        mid = (lo & hi) + ((lo ^ hi) >> 1)
        cnt = jnp.sum((keys >= mid).astype(jnp.int32), axis=1, keepdims=True)
        ge = cnt >= KS
        eq = cnt == KS
        lo = jnp.where(ge, mid, lo)
        hi = jnp.where(eq, mid + 1, jnp.where(ge, hi, mid))
        return lo, hi

    lo, _ = jax.lax.while_loop(cond, body, (lo, hi))

    mask = keys >= lo
    rowmax = jnp.max(d, axis=1, keepdims=True)
    e = jnp.where(mask, jnp.exp(d - rowmax), 0.0)
    w = e / jnp.sum(e, axis=1, keepdims=True)
    head_out = jnp.dot(w, v_ref[0], preferred_element_type=jnp.float32)
    contrib = jnp.dot(head_out, wp_ref[0], preferred_element_type=jnp.float32)

    @pl.when(h == 0)
    def _():
        out_ref[...] = contrib + bp_ref[...]

    @pl.when(h != 0)
    def _():
        out_ref[...] += contrib


def kernel(vid, Wq, bq, Wk, bk, Wv, bv, Wp, bp):
    Bv, Tv, Cv, Hv, Wd = vid.shape
    N = Tv * Hv * Wd
    dh = Cv // NHEADS
    scale = dh ** -0.5
    x = vid.transpose(0, 1, 3, 4, 2).reshape(N, Cv)

    def col_heads(w):
        return w.reshape(Cv, NHEADS, dh).transpose(1, 0, 2)

    def bias_heads(b):
        return b.reshape(NHEADS, 1, dh)

    q, k, v = pl.pallas_call(
        functools.partial(_qkv_kernel, scale=scale),
        grid=(NHEADS,),
        in_specs=[
            pl.BlockSpec((N, Cv), lambda h: (0, 0)),
            pl.BlockSpec((1, Cv, dh), lambda h: (h, 0, 0)),
            pl.BlockSpec((1, 1, dh), lambda h: (h, 0, 0)),
            pl.BlockSpec((1, Cv, dh), lambda h: (h, 0, 0)),
            pl.BlockSpec((1, 1, dh), lambda h: (h, 0, 0)),
            pl.BlockSpec((1, Cv, dh), lambda h: (h, 0, 0)),
            pl.BlockSpec((1, 1, dh), lambda h: (h, 0, 0)),
        ],
        out_specs=[pl.BlockSpec((1, N, dh), lambda h: (h, 0, 0))] * 3,
        out_shape=[jax.ShapeDtypeStruct((NHEADS, N, dh), jnp.float32)] * 3,
    )(x, col_heads(Wq), bias_heads(bq), col_heads(Wk), bias_heads(bk),
      col_heads(Wv), bias_heads(bv))

    nrb = N // ROW_BLOCK
    out = pl.pallas_call(
        _attn_kernel,
        grid=(nrb, NHEADS),
        in_specs=[
            pl.BlockSpec((1, ROW_BLOCK, dh), lambda rb, h: (h, rb, 0)),
            pl.BlockSpec((1, N, dh), lambda rb, h: (h, 0, 0)),
            pl.BlockSpec((1, N, dh), lambda rb, h: (h, 0, 0)),
            pl.BlockSpec((1, dh, Cv), lambda rb, h: (h, 0, 0)),
            pl.BlockSpec((1, Cv), lambda rb, h: (0, 0)),
        ],
        out_specs=pl.BlockSpec((ROW_BLOCK, Cv), lambda rb, h: (rb, 0)),
        out_shape=jax.ShapeDtypeStruct((N, Cv), jnp.float32),
    )(q, k, v, Wp.reshape(NHEADS, dh, Cv), bp.reshape(1, Cv))

    return out.reshape(Bv, Tv, Hv, Wd, Cv).transpose(0, 1, 4, 2, 3)
